# baseline (speedup 1.0000x reference)
import jax
import jax.numpy as jnp
from jax.experimental import pallas as pl
from jax.experimental.pallas import tpu as pltpu

_BM = 2048
_HBLK = 8


def _softmax_t(logits):
    lt = logits.T
    m = jnp.max(lt, axis=0, keepdims=True)
    e = jnp.exp(lt - m)
    return e / jnp.sum(e, axis=0, keepdims=True)


def _body(xt_ref, xb_ref, w_ref, top_ref, bot_ref):
    w = w_ref[...]
    top_ref[...] = _softmax_t(
        jnp.dot(xt_ref[...], w, preferred_element_type=jnp.float32))
    bot_ref[...] = _softmax_t(
        jnp.dot(xb_ref[...], w, preferred_element_type=jnp.float32))


@jax.jit
def kernel(x, w_gate, w_noise):
    n, d = x.shape
    _, k = w_gate.shape
    h = n // 2
    top_t, bot_t = pl.pallas_call(
        _body,
        grid=(h // _BM,),
        in_specs=[
            pl.BlockSpec((_BM, d), lambda i: (i, 0)),
            pl.BlockSpec((_BM, d), lambda i: (i + _HBLK, 0)),
            pl.BlockSpec((d, k), lambda i: (0, 0)),
        ],
        out_specs=[
            pl.BlockSpec((k, _BM), lambda i: (0, i)),
            pl.BlockSpec((k, _BM), lambda i: (0, i)),
        ],
        out_shape=[
            jax.ShapeDtypeStruct((k, h), jnp.float32),
            jax.ShapeDtypeStruct((k, h), jnp.float32),
        ],
        compiler_params=pltpu.CompilerParams(
            dimension_semantics=("arbitrary",),
        ),
    )(x, x, w_gate)
    return jnp.concatenate([top_t, bot_t], axis=1).T


# speedup vs baseline: 1.0604x; 1.0604x over previous
import jax
import jax.numpy as jnp
from jax.experimental import pallas as pl
from jax.experimental.pallas import tpu as pltpu

_BM = 4096


def _body(x_ref, w_ref, out_ref):
    logits = jnp.dot(x_ref[...], w_ref[...], preferred_element_type=jnp.float32)
    lt = logits.T
    m = jnp.max(lt, axis=0, keepdims=True)
    e = jnp.exp(lt - m)
    out_ref[...] = e / jnp.sum(e, axis=0, keepdims=True)


@jax.jit
def kernel(x, w_gate, w_noise):
    n, d = x.shape
    _, k = w_gate.shape
    out_t = pl.pallas_call(
        _body,
        grid=(n // _BM,),
        in_specs=[
            pl.BlockSpec((_BM, d), lambda i: (i, 0)),
            pl.BlockSpec((d, k), lambda i: (0, 0)),
        ],
        out_specs=pl.BlockSpec((k, _BM), lambda i: (0, i)),
        out_shape=jax.ShapeDtypeStruct((k, n), jnp.float32),
        compiler_params=pltpu.CompilerParams(
            dimension_semantics=("arbitrary",),
        ),
    )(x, w_gate)
    return out_t.T
